# baseline (speedup 1.0000x reference)
import functools

import jax
import jax.numpy as jnp
from jax import lax
from jax.experimental import pallas as pl
from jax.experimental.pallas import tpu as pltpu
from jax.experimental.pallas import tpu_sc as plsc

F_ITEMS = 64
L = 512
H = 768
P = 8
S = 6
NIDX = P * S
HV = H // 16

_info = plsc.get_sparse_core_info()
NC, NS = _info.num_cores, _info.num_subcores
NW = NC * NS
IPW = F_ITEMS // NW
NT = 3 * IPW


def _sc_body(emb, iv, ia0, ia1, ov, oa0, oa1, idx_v, rows_v, acc_v,
             gsem, osem, isem):
    wid = lax.axis_index("s") * NC + lax.axis_index("c")
    f0 = wid * IPW
    scale = jnp.float32(1.0 / NIDX)
    idx_refs = (iv, ia0, ia1)
    out_refs = (ov, oa0, oa1)
    tasks = [(a, j) for a in range(3) for j in range(IPW)]

    for t, (a, j) in enumerate(tasks):
        pltpu.async_copy(idx_refs[a].at[f0 + j], idx_v.at[t], isem)

    def drain_idx(t):
        a, j = tasks[t]
        pltpu.make_async_copy(idx_refs[a].at[f0 + j], idx_v.at[t],
                              isem).wait()

    def start(t):
        a, j = tasks[t]
        pltpu.async_copy(emb.at[f0 + j].at[idx_v.at[t]], rows_v.at[t % 3],
                         gsem.at[t % 3])

    drain_idx(0)
    start(0)
    drain_idx(1)
    start(1)
    for t, (a, j) in enumerate(tasks):
        if t + 2 < NT:
            drain_idx(t + 2)
            start(t + 2)
        pltpu.make_async_copy(emb.at[f0 + j].at[idx_v.at[t]],
                              rows_v.at[t % 3], gsem.at[t % 3]).wait()
        rv = rows_v.at[t % 3]

        def hbody(h, _):
            for u in range(2):
                sl = pl.ds(h * 32 + u * 16, 16)
                a4 = [rv[r, sl] for r in range(4)]
                for r in range(4, NIDX, 4):
                    for q in range(4):
                        a4[q] = a4[q] + rv[r + q, sl]
                acc_v[t, sl] = ((a4[0] + a4[1]) + (a4[2] + a4[3])) * scale
            return 0

        lax.fori_loop(0, HV // 2, hbody, 0)
        pltpu.async_copy(acc_v.at[t], out_refs[a].at[f0 + j], osem)
    for t, (a, j) in enumerate(tasks):
        pltpu.make_async_copy(acc_v.at[t], out_refs[a].at[f0 + j], osem).wait()


@jax.jit
def _pooled(emb, iv, ia0, ia1):
    mesh = plsc.VectorSubcoreMesh(core_axis_name="c", subcore_axis_name="s")
    row = jax.ShapeDtypeStruct((F_ITEMS, H), jnp.float32)
    return pl.kernel(
        _sc_body,
        out_type=(row, row, row),
        mesh=mesh,
        scratch_types=[
            pltpu.VMEM((NT, NIDX), jnp.int32),
            pltpu.VMEM((3, NIDX, H), jnp.float32),
            pltpu.VMEM((NT, H), jnp.float32),
            pltpu.SemaphoreType.DMA((3,)),
            pltpu.SemaphoreType.DMA,
            pltpu.SemaphoreType.DMA,
        ],
    )(emb, iv, ia0, ia1)


def kernel(token_emb, idx_V, idx_A0, idx_A1, B, N_max):
    Fdim = token_emb.shape[0]
    return _pooled(token_emb,
                   idx_V.reshape(Fdim, NIDX),
                   idx_A0.reshape(Fdim, NIDX),
                   idx_A1.reshape(Fdim, NIDX))


# speedup vs baseline: 1.1070x; 1.1070x over previous
import functools

import jax
import jax.numpy as jnp
from jax import lax
from jax.experimental import pallas as pl
from jax.experimental.pallas import tpu as pltpu
from jax.experimental.pallas import tpu_sc as plsc

F_ITEMS = 64
L = 512
H = 768
P = 8
S = 6
NIDX = P * S
HV = H // 16

_info = plsc.get_sparse_core_info()
NC, NS = _info.num_cores, _info.num_subcores
NW = NC * NS
IPW = F_ITEMS // NW
NT = 3 * IPW


def _sc_body(emb, iv, ia0, ia1, ov, oa0, oa1, idx_v, rows_v, acc_v,
             gsem, osem, isem):
    wid = lax.axis_index("s") * NC + lax.axis_index("c")
    f0 = wid * IPW
    scale = jnp.float32(1.0 / NIDX)
    idx_refs = (iv, ia0, ia1)
    out_refs = (ov, oa0, oa1)
    tasks = [(a, j) for a in range(3) for j in range(IPW)]

    for t, (a, j) in enumerate(tasks):
        pltpu.async_copy(idx_refs[a].at[f0 + j], idx_v.at[t], isem)

    def drain_idx(t):
        a, j = tasks[t]
        pltpu.make_async_copy(idx_refs[a].at[f0 + j], idx_v.at[t],
                              isem).wait()

    def start(t):
        a, j = tasks[t]
        pltpu.async_copy(emb.at[f0 + j].at[idx_v.at[t]], rows_v.at[t % 3],
                         gsem.at[t % 3])

    drain_idx(0)
    start(0)
    drain_idx(1)
    start(1)
    for t, (a, j) in enumerate(tasks):
        if t + 2 < NT:
            drain_idx(t + 2)
            start(t + 2)
        pltpu.make_async_copy(emb.at[f0 + j].at[idx_v.at[t]],
                              rows_v.at[t % 3], gsem.at[t % 3]).wait()
        rv = rows_v.at[t % 3]

        def hbody(h, _):
            sl = pl.ds(h * 16, 16)
            a4 = [rv[r, sl] for r in range(4)]
            for r in range(4, NIDX, 4):
                for q in range(4):
                    a4[q] = a4[q] + rv[r + q, sl]
            acc_v[t, sl] = ((a4[0] + a4[1]) + (a4[2] + a4[3])) * scale
            return 0

        lax.fori_loop(0, HV, hbody, 0)
        pltpu.async_copy(acc_v.at[t], out_refs[a].at[f0 + j], osem)
    for t, (a, j) in enumerate(tasks):
        pltpu.make_async_copy(acc_v.at[t], out_refs[a].at[f0 + j], osem).wait()


@jax.jit
def _pooled(emb, iv, ia0, ia1):
    mesh = plsc.VectorSubcoreMesh(core_axis_name="c", subcore_axis_name="s")
    row = jax.ShapeDtypeStruct((F_ITEMS, H), jnp.float32)
    return pl.kernel(
        _sc_body,
        out_type=(row, row, row),
        mesh=mesh,
        scratch_types=[
            pltpu.VMEM((NT, NIDX), jnp.int32),
            pltpu.VMEM((3, NIDX, H), jnp.float32),
            pltpu.VMEM((NT, H), jnp.float32),
            pltpu.SemaphoreType.DMA((3,)),
            pltpu.SemaphoreType.DMA,
            pltpu.SemaphoreType.DMA,
        ],
    )(emb, iv, ia0, ia1)


def kernel(token_emb, idx_V, idx_A0, idx_A1, B, N_max):
    Fdim = token_emb.shape[0]
    return _pooled(token_emb,
                   idx_V.reshape(Fdim, NIDX),
                   idx_A0.reshape(Fdim, NIDX),
                   idx_A1.reshape(Fdim, NIDX))


# speedup vs baseline: 1.1091x; 1.0019x over previous
import jax
import jax.numpy as jnp
from jax import lax
from jax.experimental import pallas as pl
from jax.experimental.pallas import tpu as pltpu
from jax.experimental.pallas import tpu_sc as plsc

F_ITEMS = 64
L = 512
H = 768
P = 8
S = 6
NIDX = P * S
HV = H // 16

_info = plsc.get_sparse_core_info()
NC, NS = _info.num_cores, _info.num_subcores
NW = NC * NS
IPW = F_ITEMS // NW
NT = 3 * IPW


def _sc_body(emb, iv, ia0, ia1, ov, oa0, oa1, idx_v, rows_v, acc_v,
             gsem, osem, isem):
    wid = lax.axis_index("s") * NC + lax.axis_index("c")
    f0 = wid * IPW
    scale = jnp.float32(1.0 / NIDX)
    idx_refs = (iv, ia0, ia1)
    out_refs = (ov, oa0, oa1)
    tasks = [(a, j) for a in range(3) for j in range(IPW)]

    for t, (a, j) in enumerate(tasks):
        pltpu.async_copy(idx_refs[a].at[f0 + j], idx_v.at[t], isem)

    def drain_idx(t):
        a, j = tasks[t]
        pltpu.make_async_copy(idx_refs[a].at[f0 + j], idx_v.at[t],
                              isem).wait()

    def start(t):
        a, j = tasks[t]
        pltpu.async_copy(emb.at[f0 + j].at[idx_v.at[t]], rows_v.at[t % 3],
                         gsem.at[t % 3])

    drain_idx(0)
    start(0)
    drain_idx(1)
    start(1)
    for t, (a, j) in enumerate(tasks):
        if t + 2 < NT:
            drain_idx(t + 2)
            start(t + 2)
        pltpu.make_async_copy(emb.at[f0 + j].at[idx_v.at[t]],
                              rows_v.at[t % 3], gsem.at[t % 3]).wait()
        rv = rows_v.at[t % 3]

        def hbody(h, _):
            sl = pl.ds(h * 16, 16)
            a4 = [rv[r, sl] for r in range(4)]
            for r in range(4, NIDX, 4):
                for q in range(4):
                    a4[q] = a4[q] + rv[r + q, sl]
            acc_v[t, sl] = ((a4[0] + a4[1]) + (a4[2] + a4[3])) * scale
            return 0

        lax.fori_loop(0, HV, hbody, 0)
        pltpu.async_copy(acc_v.at[t], out_refs[a].at[f0 + j], osem)
    for t, (a, j) in enumerate(tasks):
        pltpu.make_async_copy(acc_v.at[t], out_refs[a].at[f0 + j], osem).wait()


@jax.jit
def _pooled(emb, iv, ia0, ia1):
    mesh = plsc.VectorSubcoreMesh(core_axis_name="c", subcore_axis_name="s")
    row = jax.ShapeDtypeStruct((F_ITEMS, H), jnp.float32)
    return pl.kernel(
        _sc_body,
        out_type=(row, row, row),
        mesh=mesh,
        scratch_types=[
            pltpu.VMEM((NT, NIDX), jnp.int32),
            pltpu.VMEM((3, NIDX, H), jnp.float32),
            pltpu.VMEM((NT, H), jnp.float32),
            pltpu.SemaphoreType.DMA((3,)),
            pltpu.SemaphoreType.DMA,
            pltpu.SemaphoreType.DMA,
        ],
    )(emb, iv, ia0, ia1)


def kernel(token_emb, idx_V, idx_A0, idx_A1, B, N_max):
    Fdim = token_emb.shape[0]
    return _pooled(token_emb,
                   idx_V.reshape(Fdim, NIDX),
                   idx_A0.reshape(Fdim, NIDX),
                   idx_A1.reshape(Fdim, NIDX))
